# baseline (speedup 1.0000x reference)
import jax
import jax.numpy as jnp
from jax.experimental import pallas as pl
from jax.experimental.pallas import tpu as pltpu

_JB = 1024


def _chamfer_body(x1t_ref, x2t_ref, out_ref):
    b = pl.program_id(0)
    nb = pl.num_programs(0)
    x1t = x1t_ref[0]
    x2t = x2t_ref[0]
    n1 = x1t.shape[1]
    n2 = x2t.shape[1]

    a1 = x1t * (-2.0)
    yy_row = jnp.sum(x2t * x2t, axis=0, keepdims=True)
    yy_col = jnp.reshape(yy_row, (n2, 1))

    minv = jnp.full((n1,), jnp.inf, dtype=jnp.float32)
    for j in range(n2 // _JB):
        x2b = jax.lax.slice(x2t, (0, j * _JB), (3, j * _JB + _JB))
        yyb = jax.lax.slice(yy_col, (j * _JB, 0), (j * _JB + _JB, 1))
        g = jax.lax.dot_general(
            x2b, a1, (((0,), (0,)), ((), ())),
            preferred_element_type=jnp.float32)
        minv = jnp.minimum(minv, jnp.min(g + yyb, axis=0))

    xx = jnp.sum(x1t * x1t, axis=0)
    d = jnp.maximum(minv + xx, 0.0)
    s = jnp.sum(jnp.sqrt(d))

    @pl.when(b == 0)
    def _():
        out_ref[0] = 0.0

    out_ref[0] += s / (n1 * nb)


def kernel(xyz1, xyz2):
    bsz, n1, _ = xyz1.shape
    n2 = xyz2.shape[1]
    xyz1t = jnp.transpose(xyz1, (0, 2, 1))
    xyz2t = jnp.transpose(xyz2, (0, 2, 1))
    out = pl.pallas_call(
        _chamfer_body,
        grid=(bsz,),
        in_specs=[
            pl.BlockSpec((1, 3, n1), lambda b: (b, 0, 0)),
            pl.BlockSpec((1, 3, n2), lambda b: (b, 0, 0)),
        ],
        out_specs=pl.BlockSpec(memory_space=pltpu.SMEM),
        out_shape=jax.ShapeDtypeStruct((1,), jnp.float32),
        compiler_params=pltpu.CompilerParams(
            dimension_semantics=("arbitrary",),
        ),
    )(xyz1t, xyz2t)
    return out[0]
